# baseline (speedup 1.0000x reference)
import functools

import jax
import jax.numpy as jnp
from jax import lax
from jax.experimental import pallas as pl
from jax.experimental.pallas import tpu as pltpu
from jax.experimental.pallas import tpu_sc as plsc

E = 320000
H = 128
K = 5
EC = E // K

NC = 2
NS = 16
NW = NC * NS
EPW = EC // NW
C = 80
NCHUNK = EPW // C
NB = 5
NOUTER = NCHUNK // NB


def _gather_body(kc, za_hbm, zh_hbm, eli_hbm, out_hbm,
                 rowv, colv, bufa, bufh, gsa, gsh, ssa, ssh):
    wid = lax.axis_index("s") * NC + lax.axis_index("c")
    base = wid * EPW
    g0 = kc * EC + base
    pltpu.sync_copy(eli_hbm.at[pl.ds(g0, EPW)], rowv)
    pltpu.sync_copy(eli_hbm.at[pl.ds(E + g0, EPW)], colv)

    def start_gather(k, b):
        off = pl.multiple_of(k * C, 8)
        pltpu.async_copy(za_hbm.at[rowv.at[pl.ds(off, C)]], bufa.at[b], gsa.at[b])
        pltpu.async_copy(zh_hbm.at[colv.at[pl.ds(off, C)]], bufh.at[b], gsh.at[b])

    def wait_gather(b):
        pltpu.make_async_copy(za_hbm.at[rowv.at[pl.ds(0, C)]], bufa.at[b], gsa.at[b]).wait()
        pltpu.make_async_copy(zh_hbm.at[colv.at[pl.ds(0, C)]], bufh.at[b], gsh.at[b]).wait()

    def start_store(k, b):
        e0 = base + k * C
        pltpu.async_copy(bufa.at[b], out_hbm.at[pl.ds(e0, C), pl.ds(0, H)], ssa.at[b])
        pltpu.async_copy(bufh.at[b], out_hbm.at[pl.ds(e0, C), pl.ds(H, H)], ssh.at[b])

    def wait_store(b):
        pltpu.make_async_copy(bufa.at[b], out_hbm.at[pl.ds(0, C), pl.ds(0, H)], ssa.at[b]).wait()
        pltpu.make_async_copy(bufh.at[b], out_hbm.at[pl.ds(0, C), pl.ds(H, H)], ssh.at[b]).wait()

    for b in range(NB):
        start_gather(b, b)

    def outer(t, carry):
        k0 = t * NB
        for b in range(NB):
            wait_gather(b)
            start_store(k0 + b, b)
        for b in range(NB):
            @pl.when(t < NOUTER - 1)
            def _():
                wait_store(b)
                start_gather(k0 + NB + b, b)
        return carry

    lax.fori_loop(0, NOUTER, outer, 0)
    for b in range(NB):
        wait_store(b)


@functools.cache
def _gather_fn(kc):
    return functools.partial(
        pl.kernel,
        mesh=plsc.VectorSubcoreMesh(core_axis_name="c", subcore_axis_name="s"),
        out_type=jax.ShapeDtypeStruct((EC, 2 * H), jnp.float32),
        scratch_types=[
            pltpu.VMEM((EPW,), jnp.int32),
            pltpu.VMEM((EPW,), jnp.int32),
            pltpu.VMEM((NB, C, H), jnp.float32),
            pltpu.VMEM((NB, C, H), jnp.float32),
            pltpu.SemaphoreType.DMA((NB,)),
            pltpu.SemaphoreType.DMA((NB,)),
            pltpu.SemaphoreType.DMA((NB,)),
            pltpu.SemaphoreType.DMA((NB,)),
        ],
    )(functools.partial(_gather_body, kc))


BLK = 6400
GPC = EC // BLK


def _mlp_first_body(z1k_ref, w1_ref, b1_ref, w2_ref, b2_ref,
                    z1o_ref, z2_ref, z3_ref):
    i = pl.program_id(0)
    x = z1k_ref[...]
    z1o_ref[...] = x
    h = jnp.dot(x, w1_ref[...], preferred_element_type=jnp.float32)
    h = jnp.maximum(h + b1_ref[...][None, :], 0.0)
    z2_ref[...] = h
    z3row = lax.dot_general(w2_ref[...], h, (((1,), (1,)), ((), ())),
                            preferred_element_type=jnp.float32)
    z3_ref[pl.ds(i * BLK, BLK)] = z3row[0, :] + b2_ref[0]


def _mlp_next_body(z1k_ref, w1_ref, b1_ref, w2_ref, b2_ref,
                   z1a_ref, z2a_ref, z1o_ref, z2_ref, z3_ref):
    _mlp_first_body(z1k_ref, w1_ref, b1_ref, w2_ref, b2_ref,
                    z1o_ref, z2_ref, z3_ref)


def _mlp_chunk(kc, z1k, W1, b1, w2t, b2, z1f, z2f):
    base = kc * GPC
    w_specs = [
        pl.BlockSpec((2 * H, H), lambda i: (0, 0)),
        pl.BlockSpec((H,), lambda i: (0,)),
        pl.BlockSpec((1, H), lambda i: (0, 0)),
        pl.BlockSpec((1,), lambda i: (0,)),
    ]
    out_specs = [
        pl.BlockSpec((BLK, 2 * H), lambda i: (base + i, 0)),
        pl.BlockSpec((BLK, H), lambda i: (base + i, 0)),
        pl.BlockSpec((EC,), lambda i: (0,)),
    ]
    out_shape = [
        jax.ShapeDtypeStruct((E, 2 * H), jnp.float32),
        jax.ShapeDtypeStruct((E, H), jnp.float32),
        jax.ShapeDtypeStruct((EC,), jnp.float32),
    ]
    if kc == 0:
        return pl.pallas_call(
            _mlp_first_body,
            grid=(GPC,),
            in_specs=[pl.BlockSpec((BLK, 2 * H), lambda i: (i, 0))] + w_specs,
            out_specs=out_specs,
            out_shape=out_shape,
            compiler_params=pltpu.CompilerParams(
                dimension_semantics=("arbitrary",),
            ),
        )(z1k, W1, b1, w2t, b2)
    return pl.pallas_call(
        _mlp_next_body,
        grid=(GPC,),
        in_specs=[pl.BlockSpec((BLK, 2 * H), lambda i: (i, 0))] + w_specs + [
            pl.BlockSpec(memory_space=pl.ANY),
            pl.BlockSpec(memory_space=pl.ANY),
        ],
        out_specs=out_specs,
        out_shape=out_shape,
        input_output_aliases={5: 0, 6: 1},
        compiler_params=pltpu.CompilerParams(
            dimension_semantics=("arbitrary",),
        ),
    )(z1k, W1, b1, w2t, b2, z1f, z2f)


def kernel(z_author, z_hotel, edge_label_index, W1, b1, W2, b2):
    eli = edge_label_index.reshape(-1)
    w2t = W2.reshape(1, H)
    z1ks = [_gather_fn(kc)(z_author, z_hotel, eli) for kc in range(K)]
    z1f = z2f = None
    z3s = []
    for kc in range(K):
        if kc == 0:
            z1f, z2f, z3k = _mlp_chunk(0, z1ks[0], W1, b1, w2t, b2, None, None)
        else:
            z1f, z2f, z3k = _mlp_chunk(kc, z1ks[kc], W1, b1, w2t, b2, z1f, z2f)
        z3s.append(z3k)
    z3 = jnp.concatenate(z3s)
    return (z3, (z1f, z2f))


# speedup vs baseline: 1.2484x; 1.2484x over previous
import functools

import jax
import jax.numpy as jnp
from jax import lax
from jax.experimental import pallas as pl
from jax.experimental.pallas import tpu as pltpu
from jax.experimental.pallas import tpu_sc as plsc

E = 320000
H = 128

NC = 2
NS = 16
NW = NC * NS
EPW = E // NW
C = 80
NCHUNK = EPW // C
NB = 5
NOUTER = NCHUNK // NB


def _gather_body(za_hbm, zh_hbm, eli_hbm, out_hbm,
                 rowv, colv, bufa, bufh, gsa, gsh, ssa, ssh):
    wid = lax.axis_index("s") * NC + lax.axis_index("c")
    base = wid * EPW
    pltpu.sync_copy(eli_hbm.at[pl.ds(base, EPW)], rowv)
    pltpu.sync_copy(eli_hbm.at[pl.ds(E + base, EPW)], colv)

    def start_gather(k, b):
        off = pl.multiple_of(k * C, 8)
        pltpu.async_copy(za_hbm.at[rowv.at[pl.ds(off, C)]], bufa.at[b], gsa.at[b])
        pltpu.async_copy(zh_hbm.at[colv.at[pl.ds(off, C)]], bufh.at[b], gsh.at[b])

    def wait_gather(b):
        pltpu.make_async_copy(za_hbm.at[rowv.at[pl.ds(0, C)]], bufa.at[b], gsa.at[b]).wait()
        pltpu.make_async_copy(zh_hbm.at[colv.at[pl.ds(0, C)]], bufh.at[b], gsh.at[b]).wait()

    def start_store(k, b):
        e0 = base + k * C
        pltpu.async_copy(bufa.at[b], out_hbm.at[pl.ds(e0, C), pl.ds(0, H)], ssa.at[b])
        pltpu.async_copy(bufh.at[b], out_hbm.at[pl.ds(e0, C), pl.ds(H, H)], ssh.at[b])

    def wait_store(b):
        pltpu.make_async_copy(bufa.at[b], out_hbm.at[pl.ds(0, C), pl.ds(0, H)], ssa.at[b]).wait()
        pltpu.make_async_copy(bufh.at[b], out_hbm.at[pl.ds(0, C), pl.ds(H, H)], ssh.at[b]).wait()

    for b in range(NB):
        start_gather(b, b)

    def outer(t, carry):
        k0 = t * NB
        for b in range(NB):
            wait_gather(b)
            start_store(k0 + b, b)
        for b in range(NB):
            @pl.when(t < NOUTER - 1)
            def _():
                wait_store(b)
                start_gather(k0 + NB + b, b)
        return carry

    lax.fori_loop(0, NOUTER, outer, 0)
    for b in range(NB):
        wait_store(b)


@functools.cache
def _gather_fn():
    return functools.partial(
        pl.kernel,
        mesh=plsc.VectorSubcoreMesh(core_axis_name="c", subcore_axis_name="s"),
        out_type=jax.ShapeDtypeStruct((E, 2 * H), jnp.float32),
        scratch_types=[
            pltpu.VMEM((EPW,), jnp.int32),
            pltpu.VMEM((EPW,), jnp.int32),
            pltpu.VMEM((NB, C, H), jnp.float32),
            pltpu.VMEM((NB, C, H), jnp.float32),
            pltpu.SemaphoreType.DMA((NB,)),
            pltpu.SemaphoreType.DMA((NB,)),
            pltpu.SemaphoreType.DMA((NB,)),
            pltpu.SemaphoreType.DMA((NB,)),
        ],
    )(_gather_body)


BLK = 12800


def _mlp_body(z1_ref, w1_ref, b1_ref, w2_ref, b2_ref, z2_ref, z3_ref):
    i = pl.program_id(0)
    x = z1_ref[...]
    h = jnp.dot(x, w1_ref[...], preferred_element_type=jnp.float32)
    h = jnp.maximum(h + b1_ref[...][None, :], 0.0)
    z2_ref[...] = h
    z3row = lax.dot_general(w2_ref[...], h, (((1,), (1,)), ((), ())),
                            preferred_element_type=jnp.float32)
    z3_ref[pl.ds(i * BLK, BLK)] = z3row[0, :] + b2_ref[0]


def _mlp(z1, W1, b1, w2t, b2):
    grid = (E // BLK,)
    return pl.pallas_call(
        _mlp_body,
        grid=grid,
        in_specs=[
            pl.BlockSpec((BLK, 2 * H), lambda i: (i, 0)),
            pl.BlockSpec((2 * H, H), lambda i: (0, 0)),
            pl.BlockSpec((H,), lambda i: (0,)),
            pl.BlockSpec((1, H), lambda i: (0, 0)),
            pl.BlockSpec((1,), lambda i: (0,)),
        ],
        out_specs=[
            pl.BlockSpec((BLK, H), lambda i: (i, 0)),
            pl.BlockSpec((E,), lambda i: (0,)),
        ],
        out_shape=[
            jax.ShapeDtypeStruct((E, H), jnp.float32),
            jax.ShapeDtypeStruct((E,), jnp.float32),
        ],
        compiler_params=pltpu.CompilerParams(
            dimension_semantics=("arbitrary",),
        ),
    )(z1, W1, b1, w2t, b2)


def kernel(z_author, z_hotel, edge_label_index, W1, b1, W2, b2):
    z1 = _gather_fn()(z_author, z_hotel, edge_label_index.reshape(-1))
    z2, z3 = _mlp(z1, W1, b1, W2.reshape(1, H), b2)
    return (z3, (z1, z2))


# speedup vs baseline: 1.2542x; 1.0047x over previous
import functools

import jax
import jax.numpy as jnp
from jax import lax
from jax.experimental import pallas as pl
from jax.experimental.pallas import tpu as pltpu
from jax.experimental.pallas import tpu_sc as plsc

E = 320000
H = 128

NC = 2
NS = 16
NW = NC * NS
EPW = E // NW
C = 80
NCHUNK = EPW // C
NB = 5
NOUTER = NCHUNK // NB


def _gather_body(za_hbm, zh_hbm, eli_hbm, out_hbm,
                 rowv, colv, bufa, bufh, gsa, gsh, ssa, ssh):
    wid = lax.axis_index("s") * NC + lax.axis_index("c")
    base = wid * EPW
    pltpu.sync_copy(eli_hbm.at[pl.ds(base, EPW)], rowv)
    pltpu.sync_copy(eli_hbm.at[pl.ds(E + base, EPW)], colv)

    def start_gather(k, b):
        off = pl.multiple_of(k * C, 8)
        pltpu.async_copy(za_hbm.at[rowv.at[pl.ds(off, C)]], bufa.at[b], gsa.at[b])
        pltpu.async_copy(zh_hbm.at[colv.at[pl.ds(off, C)]], bufh.at[b], gsh.at[b])

    def wait_gather(b):
        pltpu.make_async_copy(za_hbm.at[rowv.at[pl.ds(0, C)]], bufa.at[b], gsa.at[b]).wait()
        pltpu.make_async_copy(zh_hbm.at[colv.at[pl.ds(0, C)]], bufh.at[b], gsh.at[b]).wait()

    def start_store(k, b):
        e0 = base + k * C
        pltpu.async_copy(bufa.at[b], out_hbm.at[pl.ds(e0, C), pl.ds(0, H)], ssa.at[b])
        pltpu.async_copy(bufh.at[b], out_hbm.at[pl.ds(e0, C), pl.ds(H, H)], ssh.at[b])

    def wait_store(b):
        pltpu.make_async_copy(bufa.at[b], out_hbm.at[pl.ds(0, C), pl.ds(0, H)], ssa.at[b]).wait()
        pltpu.make_async_copy(bufh.at[b], out_hbm.at[pl.ds(0, C), pl.ds(H, H)], ssh.at[b]).wait()

    for b in range(NB):
        start_gather(b, b)

    def outer(t, carry):
        k0 = t * NB
        for b in range(NB):
            wait_gather(b)
            start_store(k0 + b, b)
        for b in range(NB):
            @pl.when(t < NOUTER - 1)
            def _():
                wait_store(b)
                start_gather(k0 + NB + b, b)
        return carry

    lax.fori_loop(0, NOUTER, outer, 0)
    for b in range(NB):
        wait_store(b)


@functools.cache
def _gather_fn():
    return functools.partial(
        pl.kernel,
        mesh=plsc.VectorSubcoreMesh(core_axis_name="c", subcore_axis_name="s"),
        out_type=jax.ShapeDtypeStruct((E, 2 * H), jnp.float32),
        scratch_types=[
            pltpu.VMEM((EPW,), jnp.int32),
            pltpu.VMEM((EPW,), jnp.int32),
            pltpu.VMEM((NB, C, H), jnp.float32),
            pltpu.VMEM((NB, C, H), jnp.float32),
            pltpu.SemaphoreType.DMA((NB,)),
            pltpu.SemaphoreType.DMA((NB,)),
            pltpu.SemaphoreType.DMA((NB,)),
            pltpu.SemaphoreType.DMA((NB,)),
        ],
    )(_gather_body)


BLK = 16000


def _mlp_body(z1_ref, w1_ref, b1_ref, w2_ref, b2_ref, z2_ref, z3_ref):
    i = pl.program_id(0)
    x = z1_ref[...]
    h = jnp.dot(x, w1_ref[...], preferred_element_type=jnp.float32)
    h = jnp.maximum(h + b1_ref[...][None, :], 0.0)
    z2_ref[...] = h
    z3row = lax.dot_general(w2_ref[...], h, (((1,), (1,)), ((), ())),
                            preferred_element_type=jnp.float32)
    z3_ref[pl.ds(i * BLK, BLK)] = z3row[0, :] + b2_ref[0]


def _mlp(z1, W1, b1, w2t, b2):
    grid = (E // BLK,)
    return pl.pallas_call(
        _mlp_body,
        grid=grid,
        in_specs=[
            pl.BlockSpec((BLK, 2 * H), lambda i: (i, 0)),
            pl.BlockSpec((2 * H, H), lambda i: (0, 0)),
            pl.BlockSpec((H,), lambda i: (0,)),
            pl.BlockSpec((1, H), lambda i: (0, 0)),
            pl.BlockSpec((1,), lambda i: (0,)),
        ],
        out_specs=[
            pl.BlockSpec((BLK, H), lambda i: (i, 0)),
            pl.BlockSpec((E,), lambda i: (0,)),
        ],
        out_shape=[
            jax.ShapeDtypeStruct((E, H), jnp.float32),
            jax.ShapeDtypeStruct((E,), jnp.float32),
        ],
        compiler_params=pltpu.CompilerParams(
            dimension_semantics=("arbitrary",),
        ),
    )(z1, W1, b1, w2t, b2)


def kernel(z_author, z_hotel, edge_label_index, W1, b1, W2, b2):
    z1 = _gather_fn()(z_author, z_hotel, edge_label_index.reshape(-1))
    z2, z3 = _mlp(z1, W1, b1, W2.reshape(1, H), b2)
    return (z3, (z1, z2))
